# baseline (speedup 1.0000x reference)
import functools

import jax
import jax.numpy as jnp
from jax import lax
from jax.experimental import pallas as pl
from jax.experimental.pallas import tpu as pltpu
from jax.experimental.pallas import tpu_sc as plsc

N = 50000
E = 800000
HID = 64
HH = 32
CHUNK = 128
NCHUNK = E // CHUNK
NB = 5
NSB = NCHUNK // NB
NSUB = 16
NPAD = 16 * 3136
SL = 3136
XB = 200
NXB = N // XB
XB2 = 400
NXB2 = N // XB2
NSBH = NSB // 2
IN_F = 10
F1P = 16
RBLK = 2000
GRID = N // RBLK

_sc_mesh = plsc.VectorSubcoreMesh(core_axis_name="c", subcore_axis_name="s")
_sc_params = pltpu.CompilerParams(
    use_tc_tiling_on_sc=False, skip_device_barrier=True
)


def _deg_body(dst_hbm, zeros_hbm, degp_hbm, dacc, dst_blk, ones_v, buf, ssem):
    c = lax.axis_index("c")
    s = lax.axis_index("s")
    w = s * 2 + c
    sl0 = s * SL
    pltpu.sync_copy(zeros_hbm.at[pl.ds(sl0, SL)], buf)
    pltpu.sync_copy(buf, dacc.at[pl.ds(sl0, SL)])
    for i in range(CHUNK // 16):
        ones_v[pl.ds(i * 16, 16)] = jnp.full((16,), 1.0, jnp.float32)
    plsc.subcore_barrier()

    def body(k, carry):
        j = w + k * 32

        @pl.when(j < NSB)
        def _():
            @pl.when(k > 0)
            def _():
                for b in range(NB):
                    pltpu.make_async_copy(
                        ones_v, dacc.at[dst_blk.at[b]], ssem
                    ).wait()

            pltpu.sync_copy(dst_hbm.at[pl.ds(j * NB, NB)], dst_blk)
            for b in range(NB):
                pltpu.async_copy(ones_v, dacc.at[dst_blk.at[b]], ssem, add=True)

        return carry

    lax.fori_loop(0, (NSB + 31) // 32, body, 0)
    for b in range(NB):
        pltpu.make_async_copy(ones_v, dacc.at[dst_blk.at[b]], ssem).wait()
    plsc.subcore_barrier()
    pltpu.sync_copy(dacc.at[pl.ds(sl0, SL)], buf)
    pltpu.sync_copy(buf, degp_hbm.at[c, pl.ds(sl0, SL)])


_deg = functools.partial(
    pl.kernel,
    out_type=jax.ShapeDtypeStruct((2, NPAD), jnp.float32),
    mesh=_sc_mesh,
    scratch_types=[
        pltpu.VMEM_SHARED((NPAD,), jnp.float32),
        pltpu.VMEM((NB, CHUNK), jnp.int32),
        pltpu.VMEM((CHUNK,), jnp.float32),
        pltpu.VMEM((SL,), jnp.float32),
        pltpu.SemaphoreType.DMA,
    ],
    compiler_params=_sc_params,
)(_deg_body)


def _agg_body(
    g_hbm,
    srcs_hbm,
    dst_hbm,
    out_hbm,
    acc,
    src_blk,
    dst_blk,
    rows,
    xfer,
    gsem,
    ssem,
    isem0,
    isem1,
):
    c = lax.axis_index("c")
    s = lax.axis_index("s")
    for t in range((NXB + NSUB - 1) // NSUB):
        j = s + t * NSUB

        @pl.when(j < NXB)
        def _():
            r = j * XB
            pltpu.sync_copy(g_hbm.at[pl.ds(c * N + r, XB)], xfer)
            pltpu.sync_copy(xfer, acc.at[pl.ds(r, XB)])

    plsc.subcore_barrier()
    isems = (isem0, isem1)

    def fire_idx(j, slot):
        j0 = j * NB
        pltpu.async_copy(srcs_hbm.at[c, pl.ds(j0, NB)], src_blk.at[slot], isems[slot])
        pltpu.async_copy(dst_hbm.at[pl.ds(j0, NB)], dst_blk.at[slot], isems[slot])

    def wait_idx(slot):
        pltpu.make_async_copy(
            srcs_hbm.at[c, pl.ds(0, NB)], src_blk.at[slot], isems[slot]
        ).wait()
        pltpu.make_async_copy(
            dst_hbm.at[pl.ds(0, NB)], dst_blk.at[slot], isems[slot]
        ).wait()

    def drain_scatters():
        for b in range(NB):
            pltpu.make_async_copy(rows.at[b], acc.at[dst_blk.at[0, b]], ssem).wait()

    fire_idx(s, 0)

    def half(j, slot, drain_pred):
        other = 1 - slot

        @pl.when(j < NSB)
        def _():
            if drain_pred is None:
                drain_scatters()
            else:

                @pl.when(drain_pred)
                def _():
                    drain_scatters()

            wait_idx(slot)
            for b in range(NB):
                pltpu.async_copy(g_hbm.at[src_blk.at[slot, b]], rows.at[b], gsem)
            nj = j + NSUB

            @pl.when(nj < NSB)
            def _():
                fire_idx(nj, other)

            for b in range(NB):
                pltpu.make_async_copy(
                    g_hbm.at[src_blk.at[slot, b]], rows.at[b], gsem
                ).wait()
            for b in range(NB):
                pltpu.async_copy(rows.at[b], acc.at[dst_blk.at[slot, b]], ssem, add=True)

    def body(t, carry):
        half(s + (2 * t) * NSUB, 0, t > 0)
        half(s + (2 * t + 1) * NSUB, 1, None)
        return carry

    lax.fori_loop(0, ((NSB + NSUB - 1) // NSUB + 1) // 2, body, 0)
    drain_scatters()
    plsc.subcore_barrier()
    for t in range((NXB + NSUB - 1) // NSUB):
        j = s + t * NSUB

        @pl.when(j < NXB)
        def _():
            r = j * XB
            pltpu.sync_copy(acc.at[pl.ds(r, XB)], xfer)
            pltpu.sync_copy(xfer, out_hbm.at[c, pl.ds(r, XB)])


_agg = functools.partial(
    pl.kernel,
    out_type=jax.ShapeDtypeStruct((2, N, HH), jnp.float32),
    mesh=_sc_mesh,
    scratch_types=[
        pltpu.VMEM_SHARED((N, HH), jnp.float32),
        pltpu.VMEM((2, NB, CHUNK), jnp.int32),
        pltpu.VMEM((2, NB, CHUNK), jnp.int32),
        pltpu.VMEM((NB, CHUNK, HH), jnp.float32),
        pltpu.VMEM((XB, HH), jnp.float32),
        pltpu.SemaphoreType.DMA,
        pltpu.SemaphoreType.DMA,
        pltpu.SemaphoreType.DMA,
        pltpu.SemaphoreType.DMA,
    ],
    compiler_params=_sc_params,
)(_agg_body)


def _aggx_body(
    gx_hbm,
    srcs_hbm,
    dst_hbm,
    out_hbm,
    acc,
    src_blk,
    dst_blk,
    rows,
    xfer,
    gsem,
    ssem,
    isem0,
    isem1,
):
    c = lax.axis_index("c")
    s = lax.axis_index("s")
    base = c * NSBH
    for t in range((NXB2 + NSUB - 1) // NSUB):
        j = s + t * NSUB

        @pl.when(j < NXB2)
        def _():
            r = j * XB2
            pltpu.sync_copy(gx_hbm.at[pl.ds(r, XB2)], xfer)
            pltpu.sync_copy(xfer, acc.at[pl.ds(r, XB2)])

    plsc.subcore_barrier()
    isems = (isem0, isem1)

    def fire_idx(l, slot):
        j0 = (base + l) * NB
        pltpu.async_copy(srcs_hbm.at[0, pl.ds(j0, NB)], src_blk.at[slot], isems[slot])
        pltpu.async_copy(dst_hbm.at[pl.ds(j0, NB)], dst_blk.at[slot], isems[slot])

    def wait_idx(slot):
        pltpu.make_async_copy(
            srcs_hbm.at[0, pl.ds(0, NB)], src_blk.at[slot], isems[slot]
        ).wait()
        pltpu.make_async_copy(
            dst_hbm.at[pl.ds(0, NB)], dst_blk.at[slot], isems[slot]
        ).wait()

    def drain_scatters():
        for b in range(NB):
            pltpu.make_async_copy(rows.at[b], acc.at[dst_blk.at[0, b]], ssem).wait()

    fire_idx(s, 0)

    def half(l, slot, drain_pred):
        other = 1 - slot

        @pl.when(l < NSBH)
        def _():
            if drain_pred is None:
                drain_scatters()
            else:

                @pl.when(drain_pred)
                def _():
                    drain_scatters()

            wait_idx(slot)
            for b in range(NB):
                pltpu.async_copy(gx_hbm.at[src_blk.at[slot, b]], rows.at[b], gsem)
            nl = l + NSUB

            @pl.when(nl < NSBH)
            def _():
                fire_idx(nl, other)

            for b in range(NB):
                pltpu.make_async_copy(
                    gx_hbm.at[src_blk.at[slot, b]], rows.at[b], gsem
                ).wait()
            for b in range(NB):
                pltpu.async_copy(rows.at[b], acc.at[dst_blk.at[slot, b]], ssem, add=True)

    def body(t, carry):
        half(s + (2 * t) * NSUB, 0, t > 0)
        half(s + (2 * t + 1) * NSUB, 1, None)
        return carry

    lax.fori_loop(0, ((NSBH + NSUB - 1) // NSUB + 1) // 2, body, 0)
    drain_scatters()
    plsc.subcore_barrier()
    for t in range((NXB2 + NSUB - 1) // NSUB):
        j = s + t * NSUB

        @pl.when(j < NXB2)
        def _():
            r = j * XB2
            pltpu.sync_copy(acc.at[pl.ds(r, XB2)], xfer)
            pltpu.sync_copy(xfer, out_hbm.at[c, pl.ds(r, XB2)])


_aggx = functools.partial(
    pl.kernel,
    out_type=jax.ShapeDtypeStruct((2, N, F1P), jnp.float32),
    mesh=_sc_mesh,
    scratch_types=[
        pltpu.VMEM_SHARED((N, F1P), jnp.float32),
        pltpu.VMEM((2, NB, CHUNK), jnp.int32),
        pltpu.VMEM((2, NB, CHUNK), jnp.int32),
        pltpu.VMEM((NB, CHUNK, F1P), jnp.float32),
        pltpu.VMEM((XB2, F1P), jnp.float32),
        pltpu.SemaphoreType.DMA,
        pltpu.SemaphoreType.DMA,
        pltpu.SemaphoreType.DMA,
        pltpu.SemaphoreType.DMA,
    ],
    compiler_params=_sc_params,
)(_aggx_body)


def _pre1_body(d0_ref, d1_ref, x_ref, gx_ref, dinv_ref):
    dinv = lax.rsqrt(d0_ref[...] + d1_ref[...] + 1.0)
    dinv_ref[...] = dinv
    gx_ref[...] = jnp.concatenate(
        [dinv * x_ref[...], jnp.zeros((RBLK, F1P - IN_F), jnp.float32)], axis=1
    )


def _mixA_body(p_ref, gx_ref, dinv_ref, w1_ref, b1_ref, w2_ref, g_ref):
    dinv = dinv_ref[...]
    y = dinv * (p_ref[0] + p_ref[1] - gx_ref[...])
    t0 = jnp.dot(y, w1_ref[0], preferred_element_type=jnp.float32) + b1_ref[0:1, :]
    h0 = jnp.maximum(t0, 0.01 * t0)
    t1 = jnp.dot(y, w1_ref[1], preferred_element_type=jnp.float32) + b1_ref[1:2, :]
    h1 = jnp.maximum(t1, 0.01 * t1)
    g_ref[0] = dinv * (
        jnp.dot(h0, w2_ref[0, 0], preferred_element_type=jnp.float32)
        + jnp.dot(h1, w2_ref[1, 0], preferred_element_type=jnp.float32)
    )
    g_ref[1] = dinv * (
        jnp.dot(h0, w2_ref[0, 1], preferred_element_type=jnp.float32)
        + jnp.dot(h1, w2_ref[1, 1], preferred_element_type=jnp.float32)
    )


def _mid_body(acc_ref, dinv_ref, b_ref, w_ref, g_ref):
    dinv = dinv_ref[...]
    t0 = dinv * acc_ref[0] + b_ref[0:1, :]
    h0 = jnp.maximum(t0, 0.01 * t0)
    t1 = dinv * acc_ref[1] + b_ref[1:2, :]
    h1 = jnp.maximum(t1, 0.01 * t1)
    y0 = jnp.dot(h0, w_ref[0, 0], preferred_element_type=jnp.float32) + jnp.dot(
        h1, w_ref[1, 0], preferred_element_type=jnp.float32
    )
    y1 = jnp.dot(h0, w_ref[0, 1], preferred_element_type=jnp.float32) + jnp.dot(
        h1, w_ref[1, 1], preferred_element_type=jnp.float32
    )
    g_ref[0] = dinv * y0
    g_ref[1] = dinv * y1


def _fin_body(acc_ref, dinv_ref, b_ref, wfc_ref, bfc_ref, out_ref):
    dinv = dinv_ref[...]
    t0 = dinv * acc_ref[0] + b_ref[0:1, :]
    h0 = jnp.maximum(t0, 0.01 * t0)
    t1 = dinv * acc_ref[1] + b_ref[1:2, :]
    h1 = jnp.maximum(t1, 0.01 * t1)
    out_ref[...] = (
        jnp.dot(h0, wfc_ref[0], preferred_element_type=jnp.float32)
        + jnp.dot(h1, wfc_ref[1], preferred_element_type=jnp.float32)
        + bfc_ref[0:1, :]
    )


def _pre1(d0, d1, x):
    return pl.pallas_call(
        _pre1_body,
        grid=(GRID,),
        in_specs=[
            pl.BlockSpec((RBLK, 1), lambda i: (i, 0)),
            pl.BlockSpec((RBLK, 1), lambda i: (i, 0)),
            pl.BlockSpec((RBLK, IN_F), lambda i: (i, 0)),
        ],
        out_specs=[
            pl.BlockSpec((RBLK, F1P), lambda i: (i, 0)),
            pl.BlockSpec((RBLK, 1), lambda i: (i, 0)),
        ],
        out_shape=[
            jax.ShapeDtypeStruct((N, F1P), jnp.float32),
            jax.ShapeDtypeStruct((N, 1), jnp.float32),
        ],
    )(d0, d1, x)


def _mixA(px, gx, dinv2d, w1s, b1s, w2s):
    return pl.pallas_call(
        _mixA_body,
        grid=(GRID,),
        in_specs=[
            pl.BlockSpec((2, RBLK, F1P), lambda i: (0, i, 0)),
            pl.BlockSpec((RBLK, F1P), lambda i: (i, 0)),
            pl.BlockSpec((RBLK, 1), lambda i: (i, 0)),
            pl.BlockSpec((2, F1P, HH), lambda i: (0, 0, 0)),
            pl.BlockSpec((2, HH), lambda i: (0, 0)),
            pl.BlockSpec((2, 2, HH, HH), lambda i: (0, 0, 0, 0)),
        ],
        out_specs=pl.BlockSpec((2, RBLK, HH), lambda i: (0, i, 0)),
        out_shape=jax.ShapeDtypeStruct((2, N, HH), jnp.float32),
    )(px, gx, dinv2d, w1s, b1s, w2s)


def _mid(acc, dinv2d, b2, ws):
    return pl.pallas_call(
        _mid_body,
        grid=(GRID,),
        in_specs=[
            pl.BlockSpec((2, RBLK, HH), lambda i: (0, i, 0)),
            pl.BlockSpec((RBLK, 1), lambda i: (i, 0)),
            pl.BlockSpec((2, HH), lambda i: (0, 0)),
            pl.BlockSpec((2, 2, HH, HH), lambda i: (0, 0, 0, 0)),
        ],
        out_specs=pl.BlockSpec((2, RBLK, HH), lambda i: (0, i, 0)),
        out_shape=jax.ShapeDtypeStruct((2, N, HH), jnp.float32),
    )(acc, dinv2d, b2, ws)


def _fin(acc, dinv2d, b2, wfcs, bfc2):
    return pl.pallas_call(
        _fin_body,
        grid=(GRID,),
        in_specs=[
            pl.BlockSpec((2, RBLK, HH), lambda i: (0, i, 0)),
            pl.BlockSpec((RBLK, 1), lambda i: (i, 0)),
            pl.BlockSpec((2, HH), lambda i: (0, 0)),
            pl.BlockSpec((2, HH, 4), lambda i: (0, 0, 0)),
            pl.BlockSpec((1, 4), lambda i: (0, 0)),
        ],
        out_specs=pl.BlockSpec((RBLK, 4), lambda i: (i, 0)),
        out_shape=jax.ShapeDtypeStruct((N, 4), jnp.float32),
    )(acc, dinv2d, b2, wfcs, bfc2)


def kernel(x, edge_index, W1, b1, W2, b2, W3, b3, Wfc, bfc):
    src = edge_index[0].astype(jnp.int32)
    dst = edge_index[1].astype(jnp.int32)
    srcs3 = jnp.stack([src, src + N]).reshape(2, NCHUNK, CHUNK)
    dst3 = dst.reshape(NCHUNK, CHUNK)
    zeros = jnp.zeros((NPAD,), jnp.float32)

    degp = _deg(dst3, zeros)
    d0 = degp[0, :N].reshape(N, 1)
    d1 = degp[1, :N].reshape(N, 1)

    W1p = jnp.concatenate([W1, jnp.zeros((F1P - IN_F, HID), W1.dtype)], axis=0)
    w1s = jnp.stack([W1p[:, :HH], W1p[:, HH:]])
    w2s = jnp.stack(
        [
            jnp.stack([W2[:HH, :HH], W2[:HH, HH:]]),
            jnp.stack([W2[HH:, :HH], W2[HH:, HH:]]),
        ]
    )
    w3s = jnp.stack(
        [
            jnp.stack([W3[:HH, :HH], W3[:HH, HH:]]),
            jnp.stack([W3[HH:, :HH], W3[HH:, HH:]]),
        ]
    )
    wfcs = jnp.stack([Wfc[:HH], Wfc[HH:]])
    b1s = b1.reshape(2, HH)
    b2s = b2.reshape(2, HH)
    b3s = b3.reshape(2, HH)
    bfc2 = bfc.reshape(1, 4)

    gx, dinv2d = _pre1(d0, d1, x)
    px = _aggx(gx, srcs3, dst3)
    g = _mixA(px, gx, dinv2d, w1s, b1s, w2s)
    acc = _agg(g.reshape(2 * N, HH), srcs3, dst3)
    g = _mid(acc, dinv2d, b2s, w3s)
    acc = _agg(g.reshape(2 * N, HH), srcs3, dst3)
    return _fin(acc, dinv2d, b3s, wfcs, bfc2)


# speedup vs baseline: 1.0119x; 1.0119x over previous
import functools

import jax
import jax.numpy as jnp
from jax import lax
from jax.experimental import pallas as pl
from jax.experimental.pallas import tpu as pltpu
from jax.experimental.pallas import tpu_sc as plsc

N = 50000
E = 800000
HID = 64
HH = 32
CHUNK = 128
NCHUNK = E // CHUNK
NB = 5
NSB = NCHUNK // NB
NSUB = 16
NPAD = 16 * 3136
SL = 3136
XR = 80
NXR = N // XR
XB2 = 400
NXB2 = N // XB2
NSBH = NSB // 2
IN_F = 10
F1P = 16
RBLK = 2000
GRID = N // RBLK

_sc_mesh = plsc.VectorSubcoreMesh(core_axis_name="c", subcore_axis_name="s")
_sc_params = pltpu.CompilerParams(use_tc_tiling_on_sc=False)


def _deg_body(dst_hbm, zeros_hbm, degp_hbm, dacc, dst_blk, ones_v, buf, ssem):
    c = lax.axis_index("c")
    s = lax.axis_index("s")
    w = s * 2 + c
    sl0 = s * SL
    pltpu.sync_copy(zeros_hbm.at[pl.ds(sl0, SL)], buf)
    pltpu.sync_copy(buf, dacc.at[pl.ds(sl0, SL)])
    for i in range(CHUNK // 16):
        ones_v[pl.ds(i * 16, 16)] = jnp.full((16,), 1.0, jnp.float32)
    plsc.subcore_barrier()

    def body(k, carry):
        j = w + k * 32

        @pl.when(j < NSB)
        def _():
            @pl.when(k > 0)
            def _():
                for b in range(NB):
                    pltpu.make_async_copy(
                        ones_v, dacc.at[dst_blk.at[b]], ssem
                    ).wait()

            pltpu.sync_copy(dst_hbm.at[pl.ds(j * NB, NB)], dst_blk)
            for b in range(NB):
                pltpu.async_copy(ones_v, dacc.at[dst_blk.at[b]], ssem, add=True)

        return carry

    lax.fori_loop(0, (NSB + 31) // 32, body, 0)
    for b in range(NB):
        pltpu.make_async_copy(ones_v, dacc.at[dst_blk.at[b]], ssem).wait()
    plsc.subcore_barrier()
    pltpu.sync_copy(dacc.at[pl.ds(sl0, SL)], buf)
    pltpu.sync_copy(buf, degp_hbm.at[c, pl.ds(sl0, SL)])


_deg = functools.partial(
    pl.kernel,
    out_type=jax.ShapeDtypeStruct((2, NPAD), jnp.float32),
    mesh=_sc_mesh,
    scratch_types=[
        pltpu.VMEM_SHARED((NPAD,), jnp.float32),
        pltpu.VMEM((NB, CHUNK), jnp.int32),
        pltpu.VMEM((CHUNK,), jnp.float32),
        pltpu.VMEM((SL,), jnp.float32),
        pltpu.SemaphoreType.DMA,
    ],
    compiler_params=_sc_params,
)(_deg_body)


def _agg_body(
    g_hbm,
    srcs_hbm,
    dst_hbm,
    out_hbm,
    acc,
    src_blk,
    dst_blk,
    rows,
    gsem,
    ssem,
    isem0,
    isem1,
):
    c = lax.axis_index("c")
    s = lax.axis_index("s")

    def _ring_copy(mk_src, mk_dst, mk_src2, mk_dst2):
        for grp in range((NXR // NSUB + NB) // NB):
            js = [s + (grp * NB + b) * NSUB for b in range(NB)]

            def guarded(j, fn):
                @pl.when(j < NXR)
                def _():
                    fn()

            for b, j in enumerate(js):
                guarded(j, lambda b=b, j=j: pltpu.async_copy(mk_src(j), mk_dst(b), gsem))
            for b, j in enumerate(js):
                guarded(
                    j, lambda b=b, j=j: pltpu.make_async_copy(mk_src(j), mk_dst(b), gsem).wait()
                )
            for b, j in enumerate(js):
                guarded(j, lambda b=b, j=j: pltpu.async_copy(mk_src2(b), mk_dst2(j), ssem))
            for b, j in enumerate(js):
                guarded(
                    j,
                    lambda b=b, j=j: pltpu.make_async_copy(mk_src2(b), mk_dst2(j), ssem).wait(),
                )

    _ring_copy(
        lambda j: g_hbm.at[pl.ds(c * N + j * XR, XR)],
        lambda b: rows.at[b, pl.ds(0, XR)],
        lambda b: rows.at[b, pl.ds(0, XR)],
        lambda j: acc.at[pl.ds(j * XR, XR)],
    )
    plsc.subcore_barrier()
    isems = (isem0, isem1)

    def fire_idx(j, slot):
        j0 = j * NB
        pltpu.async_copy(srcs_hbm.at[c, pl.ds(j0, NB)], src_blk.at[slot], isems[slot])
        pltpu.async_copy(dst_hbm.at[pl.ds(j0, NB)], dst_blk.at[slot], isems[slot])

    def wait_idx(slot):
        pltpu.make_async_copy(
            srcs_hbm.at[c, pl.ds(0, NB)], src_blk.at[slot], isems[slot]
        ).wait()
        pltpu.make_async_copy(
            dst_hbm.at[pl.ds(0, NB)], dst_blk.at[slot], isems[slot]
        ).wait()

    def drain_scatters():
        for b in range(NB):
            pltpu.make_async_copy(rows.at[b], acc.at[dst_blk.at[0, b]], ssem).wait()

    fire_idx(s, 0)

    def half(j, slot, drain_pred):
        other = 1 - slot

        @pl.when(j < NSB)
        def _():
            if drain_pred is None:
                drain_scatters()
            else:

                @pl.when(drain_pred)
                def _():
                    drain_scatters()

            wait_idx(slot)
            for b in range(NB):
                pltpu.async_copy(g_hbm.at[src_blk.at[slot, b]], rows.at[b], gsem)
            nj = j + NSUB

            @pl.when(nj < NSB)
            def _():
                fire_idx(nj, other)

            for b in range(NB):
                pltpu.make_async_copy(
                    g_hbm.at[src_blk.at[slot, b]], rows.at[b], gsem
                ).wait()
            for b in range(NB):
                pltpu.async_copy(rows.at[b], acc.at[dst_blk.at[slot, b]], ssem, add=True)

    def body(t, carry):
        half(s + (2 * t) * NSUB, 0, t > 0)
        half(s + (2 * t + 1) * NSUB, 1, None)
        return carry

    lax.fori_loop(0, ((NSB + NSUB - 1) // NSUB + 1) // 2, body, 0)
    drain_scatters()
    plsc.subcore_barrier()
    _ring_copy(
        lambda j: acc.at[pl.ds(j * XR, XR)],
        lambda b: rows.at[b, pl.ds(0, XR)],
        lambda b: rows.at[b, pl.ds(0, XR)],
        lambda j: out_hbm.at[c, pl.ds(j * XR, XR)],
    )


_agg = functools.partial(
    pl.kernel,
    out_type=jax.ShapeDtypeStruct((2, N, HH), jnp.float32),
    mesh=_sc_mesh,
    scratch_types=[
        pltpu.VMEM_SHARED((N, HH), jnp.float32),
        pltpu.VMEM((2, NB, CHUNK), jnp.int32),
        pltpu.VMEM((2, NB, CHUNK), jnp.int32),
        pltpu.VMEM((NB, CHUNK, HH), jnp.float32),
        pltpu.SemaphoreType.DMA,
        pltpu.SemaphoreType.DMA,
        pltpu.SemaphoreType.DMA,
        pltpu.SemaphoreType.DMA,
    ],
    compiler_params=_sc_params,
)(_agg_body)


def _aggx_body(
    gx_hbm,
    srcs_hbm,
    dst_hbm,
    out_hbm,
    acc,
    src_blk,
    dst_blk,
    rows,
    xfer,
    gsem,
    ssem,
    isem0,
    isem1,
):
    c = lax.axis_index("c")
    s = lax.axis_index("s")
    base = c * NSBH
    for t in range((NXB2 + NSUB - 1) // NSUB):
        j = s + t * NSUB

        @pl.when(j < NXB2)
        def _():
            r = j * XB2
            pltpu.sync_copy(gx_hbm.at[pl.ds(r, XB2)], xfer)
            pltpu.sync_copy(xfer, acc.at[pl.ds(r, XB2)])

    plsc.subcore_barrier()
    isems = (isem0, isem1)

    def fire_idx(l, slot):
        j0 = (base + l) * NB
        pltpu.async_copy(srcs_hbm.at[0, pl.ds(j0, NB)], src_blk.at[slot], isems[slot])
        pltpu.async_copy(dst_hbm.at[pl.ds(j0, NB)], dst_blk.at[slot], isems[slot])

    def wait_idx(slot):
        pltpu.make_async_copy(
            srcs_hbm.at[0, pl.ds(0, NB)], src_blk.at[slot], isems[slot]
        ).wait()
        pltpu.make_async_copy(
            dst_hbm.at[pl.ds(0, NB)], dst_blk.at[slot], isems[slot]
        ).wait()

    def drain_scatters():
        for b in range(NB):
            pltpu.make_async_copy(rows.at[b], acc.at[dst_blk.at[0, b]], ssem).wait()

    fire_idx(s, 0)

    def half(l, slot, drain_pred):
        other = 1 - slot

        @pl.when(l < NSBH)
        def _():
            if drain_pred is None:
                drain_scatters()
            else:

                @pl.when(drain_pred)
                def _():
                    drain_scatters()

            wait_idx(slot)
            for b in range(NB):
                pltpu.async_copy(gx_hbm.at[src_blk.at[slot, b]], rows.at[b], gsem)
            nl = l + NSUB

            @pl.when(nl < NSBH)
            def _():
                fire_idx(nl, other)

            for b in range(NB):
                pltpu.make_async_copy(
                    gx_hbm.at[src_blk.at[slot, b]], rows.at[b], gsem
                ).wait()
            for b in range(NB):
                pltpu.async_copy(rows.at[b], acc.at[dst_blk.at[slot, b]], ssem, add=True)

    def body(t, carry):
        half(s + (2 * t) * NSUB, 0, t > 0)
        half(s + (2 * t + 1) * NSUB, 1, None)
        return carry

    lax.fori_loop(0, ((NSBH + NSUB - 1) // NSUB + 1) // 2, body, 0)
    drain_scatters()
    plsc.subcore_barrier()
    for t in range((NXB2 + NSUB - 1) // NSUB):
        j = s + t * NSUB

        @pl.when(j < NXB2)
        def _():
            r = j * XB2
            pltpu.sync_copy(acc.at[pl.ds(r, XB2)], xfer)
            pltpu.sync_copy(xfer, out_hbm.at[c, pl.ds(r, XB2)])


_aggx = functools.partial(
    pl.kernel,
    out_type=jax.ShapeDtypeStruct((2, N, F1P), jnp.float32),
    mesh=_sc_mesh,
    scratch_types=[
        pltpu.VMEM_SHARED((N, F1P), jnp.float32),
        pltpu.VMEM((2, NB, CHUNK), jnp.int32),
        pltpu.VMEM((2, NB, CHUNK), jnp.int32),
        pltpu.VMEM((NB, CHUNK, F1P), jnp.float32),
        pltpu.VMEM((XB2, F1P), jnp.float32),
        pltpu.SemaphoreType.DMA,
        pltpu.SemaphoreType.DMA,
        pltpu.SemaphoreType.DMA,
        pltpu.SemaphoreType.DMA,
    ],
    compiler_params=_sc_params,
)(_aggx_body)


def _pre1_body(d0_ref, d1_ref, x_ref, gx_ref, dinv_ref):
    dinv = lax.rsqrt(d0_ref[...] + d1_ref[...] + 1.0)
    dinv_ref[...] = dinv
    gx_ref[...] = jnp.concatenate(
        [dinv * x_ref[...], jnp.zeros((RBLK, F1P - IN_F), jnp.float32)], axis=1
    )


def _mixA_body(p_ref, gx_ref, dinv_ref, w1_ref, b1_ref, w2_ref, g_ref):
    dinv = dinv_ref[...]
    y = dinv * (p_ref[0] + p_ref[1] - gx_ref[...])
    t0 = jnp.dot(y, w1_ref[0], preferred_element_type=jnp.float32) + b1_ref[0:1, :]
    h0 = jnp.maximum(t0, 0.01 * t0)
    t1 = jnp.dot(y, w1_ref[1], preferred_element_type=jnp.float32) + b1_ref[1:2, :]
    h1 = jnp.maximum(t1, 0.01 * t1)
    g_ref[0] = dinv * (
        jnp.dot(h0, w2_ref[0, 0], preferred_element_type=jnp.float32)
        + jnp.dot(h1, w2_ref[1, 0], preferred_element_type=jnp.float32)
    )
    g_ref[1] = dinv * (
        jnp.dot(h0, w2_ref[0, 1], preferred_element_type=jnp.float32)
        + jnp.dot(h1, w2_ref[1, 1], preferred_element_type=jnp.float32)
    )


def _mid_body(acc_ref, dinv_ref, b_ref, w_ref, g_ref):
    dinv = dinv_ref[...]
    t0 = dinv * acc_ref[0] + b_ref[0:1, :]
    h0 = jnp.maximum(t0, 0.01 * t0)
    t1 = dinv * acc_ref[1] + b_ref[1:2, :]
    h1 = jnp.maximum(t1, 0.01 * t1)
    y0 = jnp.dot(h0, w_ref[0, 0], preferred_element_type=jnp.float32) + jnp.dot(
        h1, w_ref[1, 0], preferred_element_type=jnp.float32
    )
    y1 = jnp.dot(h0, w_ref[0, 1], preferred_element_type=jnp.float32) + jnp.dot(
        h1, w_ref[1, 1], preferred_element_type=jnp.float32
    )
    g_ref[0] = dinv * y0
    g_ref[1] = dinv * y1


def _fin_body(acc_ref, dinv_ref, b_ref, wfc_ref, bfc_ref, out_ref):
    dinv = dinv_ref[...]
    t0 = dinv * acc_ref[0] + b_ref[0:1, :]
    h0 = jnp.maximum(t0, 0.01 * t0)
    t1 = dinv * acc_ref[1] + b_ref[1:2, :]
    h1 = jnp.maximum(t1, 0.01 * t1)
    out_ref[...] = (
        jnp.dot(h0, wfc_ref[0], preferred_element_type=jnp.float32)
        + jnp.dot(h1, wfc_ref[1], preferred_element_type=jnp.float32)
        + bfc_ref[0:1, :]
    )


def _pre1(d0, d1, x):
    return pl.pallas_call(
        _pre1_body,
        grid=(GRID,),
        in_specs=[
            pl.BlockSpec((RBLK, 1), lambda i: (i, 0)),
            pl.BlockSpec((RBLK, 1), lambda i: (i, 0)),
            pl.BlockSpec((RBLK, IN_F), lambda i: (i, 0)),
        ],
        out_specs=[
            pl.BlockSpec((RBLK, F1P), lambda i: (i, 0)),
            pl.BlockSpec((RBLK, 1), lambda i: (i, 0)),
        ],
        out_shape=[
            jax.ShapeDtypeStruct((N, F1P), jnp.float32),
            jax.ShapeDtypeStruct((N, 1), jnp.float32),
        ],
    )(d0, d1, x)


def _mixA(px, gx, dinv2d, w1s, b1s, w2s):
    return pl.pallas_call(
        _mixA_body,
        grid=(GRID,),
        in_specs=[
            pl.BlockSpec((2, RBLK, F1P), lambda i: (0, i, 0)),
            pl.BlockSpec((RBLK, F1P), lambda i: (i, 0)),
            pl.BlockSpec((RBLK, 1), lambda i: (i, 0)),
            pl.BlockSpec((2, F1P, HH), lambda i: (0, 0, 0)),
            pl.BlockSpec((2, HH), lambda i: (0, 0)),
            pl.BlockSpec((2, 2, HH, HH), lambda i: (0, 0, 0, 0)),
        ],
        out_specs=pl.BlockSpec((2, RBLK, HH), lambda i: (0, i, 0)),
        out_shape=jax.ShapeDtypeStruct((2, N, HH), jnp.float32),
    )(px, gx, dinv2d, w1s, b1s, w2s)


def _mid(acc, dinv2d, b2, ws):
    return pl.pallas_call(
        _mid_body,
        grid=(GRID,),
        in_specs=[
            pl.BlockSpec((2, RBLK, HH), lambda i: (0, i, 0)),
            pl.BlockSpec((RBLK, 1), lambda i: (i, 0)),
            pl.BlockSpec((2, HH), lambda i: (0, 0)),
            pl.BlockSpec((2, 2, HH, HH), lambda i: (0, 0, 0, 0)),
        ],
        out_specs=pl.BlockSpec((2, RBLK, HH), lambda i: (0, i, 0)),
        out_shape=jax.ShapeDtypeStruct((2, N, HH), jnp.float32),
    )(acc, dinv2d, b2, ws)


def _fin(acc, dinv2d, b2, wfcs, bfc2):
    return pl.pallas_call(
        _fin_body,
        grid=(GRID,),
        in_specs=[
            pl.BlockSpec((2, RBLK, HH), lambda i: (0, i, 0)),
            pl.BlockSpec((RBLK, 1), lambda i: (i, 0)),
            pl.BlockSpec((2, HH), lambda i: (0, 0)),
            pl.BlockSpec((2, HH, 4), lambda i: (0, 0, 0)),
            pl.BlockSpec((1, 4), lambda i: (0, 0)),
        ],
        out_specs=pl.BlockSpec((RBLK, 4), lambda i: (i, 0)),
        out_shape=jax.ShapeDtypeStruct((N, 4), jnp.float32),
    )(acc, dinv2d, b2, wfcs, bfc2)


def kernel(x, edge_index, W1, b1, W2, b2, W3, b3, Wfc, bfc):
    src = edge_index[0].astype(jnp.int32)
    dst = edge_index[1].astype(jnp.int32)
    srcs3 = jnp.stack([src, src + N]).reshape(2, NCHUNK, CHUNK)
    dst3 = dst.reshape(NCHUNK, CHUNK)
    zeros = jnp.zeros((NPAD,), jnp.float32)

    degp = _deg(dst3, zeros)
    d0 = degp[0, :N].reshape(N, 1)
    d1 = degp[1, :N].reshape(N, 1)

    W1p = jnp.concatenate([W1, jnp.zeros((F1P - IN_F, HID), W1.dtype)], axis=0)
    w1s = jnp.stack([W1p[:, :HH], W1p[:, HH:]])
    w2s = jnp.stack(
        [
            jnp.stack([W2[:HH, :HH], W2[:HH, HH:]]),
            jnp.stack([W2[HH:, :HH], W2[HH:, HH:]]),
        ]
    )
    w3s = jnp.stack(
        [
            jnp.stack([W3[:HH, :HH], W3[:HH, HH:]]),
            jnp.stack([W3[HH:, :HH], W3[HH:, HH:]]),
        ]
    )
    wfcs = jnp.stack([Wfc[:HH], Wfc[HH:]])
    b1s = b1.reshape(2, HH)
    b2s = b2.reshape(2, HH)
    b3s = b3.reshape(2, HH)
    bfc2 = bfc.reshape(1, 4)

    gx, dinv2d = _pre1(d0, d1, x)
    px = _aggx(gx, srcs3, dst3)
    g = _mixA(px, gx, dinv2d, w1s, b1s, w2s)
    acc = _agg(g.reshape(2 * N, HH), srcs3, dst3)
    g = _mid(acc, dinv2d, b2s, w3s)
    acc = _agg(g.reshape(2 * N, HH), srcs3, dst3)
    return _fin(acc, dinv2d, b3s, wfcs, bfc2)
